# Initial kernel scaffold; baseline (speedup 1.0000x reference)
import jax
import jax.numpy as jnp
from jax.experimental import pallas as pl


def kernel(gender_emb, college_emb, school_emb, mbti_emb, final_fc_w, final_fc_b, weight, userA_gender, userA_school, userA_college, userA_mbti, userB_gender, userB_school, userB_college, userB_mbti):
    raise NotImplementedError("write your pallas kernel here")


# speedup vs baseline: 8.2682x; 8.2682x over previous
import jax
import jax.numpy as jnp
from jax.experimental import pallas as pl
from jax.experimental.pallas import tpu as pltpu

_ROWS_PER_BLOCK = 512


def _pairwise_cos(emb, eps=1e-8):
    e = emb.astype(jnp.float32)
    dots = e @ e.T
    norms = jnp.sqrt(jnp.sum(e * e, axis=-1))
    return dots / jnp.maximum(norms[:, None] * norms[None, :], eps)


def _lookup_body(tbl_ref, ag_ref, asc_ref, aco_ref, am_ref,
                 bg_ref, bsc_ref, bco_ref, bm_ref, out_ref):
    rb = out_ref.shape[0]
    rep = rb // 8
    small = pltpu.repeat(tbl_ref[0:8, :], rep, axis=0)
    m0 = pltpu.repeat(tbl_ref[8:16, :], rep, axis=0)
    m1 = pltpu.repeat(tbl_ref[16:24, :], rep, axis=0)
    m2 = pltpu.repeat(tbl_ref[24:32, :], rep, axis=0)

    kg = ag_ref[...] * 2 + bg_ref[...]
    kc = aco_ref[...] * 7 + bco_ref[...] + 4
    ks = asc_ref[...] * 8 + bsc_ref[...] + 53
    km = am_ref[...] * 17 + bm_ref[...]
    kw = km & 127

    vg = jnp.take_along_axis(small, kg, axis=1)
    vc = jnp.take_along_axis(small, kc, axis=1)
    vs = jnp.take_along_axis(small, ks, axis=1)
    u0 = jnp.take_along_axis(m0, kw, axis=1)
    u1 = jnp.take_along_axis(m1, kw, axis=1)
    u2 = jnp.take_along_axis(m2, kw, axis=1)
    vm = jnp.where(km < 128, u0, jnp.where(km < 256, u1, u2))

    out_ref[...] = jax.nn.sigmoid(vg + vc + vs + vm)


def kernel(gender_emb, college_emb, school_emb, mbti_emb, final_fc_w,
           final_fc_b, weight,
           userA_gender, userA_school, userA_college, userA_mbti,
           userB_gender, userB_school, userB_college, userB_mbti):
    w_eff = (weight.astype(jnp.float32) *
             final_fc_w.reshape(4).astype(jnp.float32))
    bias = final_fc_b.astype(jnp.float32).reshape(())

    tg = (_pairwise_cos(gender_emb) * w_eff[0] + bias).reshape(-1)
    tc = (_pairwise_cos(college_emb) * w_eff[1]).reshape(-1)
    ts = (_pairwise_cos(school_emb) * w_eff[2]).reshape(-1)
    tm = (_pairwise_cos(mbti_emb) * w_eff[3]).reshape(-1)

    small = jnp.concatenate([tg, tc, ts, jnp.zeros((11,), jnp.float32)])
    mpad = jnp.concatenate([tm, jnp.zeros((384 - 289,), jnp.float32)])
    mchunks = mpad.reshape(3, 128)
    tbl = jnp.concatenate([
        jnp.tile(small[None, :], (8, 1)),
        jnp.tile(mchunks[0][None, :], (8, 1)),
        jnp.tile(mchunks[1][None, :], (8, 1)),
        jnp.tile(mchunks[2][None, :], (8, 1)),
    ], axis=0)

    B = userA_gender.shape[0]
    rb = _ROWS_PER_BLOCK
    rows = pl.cdiv(B, 128)
    rows = pl.cdiv(rows, rb) * rb
    bpad = rows * 128 - B

    def to2d(x):
        x = x.astype(jnp.int32)
        if bpad:
            x = jnp.pad(x, (0, bpad))
        return x.reshape(rows, 128)

    idxs = [to2d(x) for x in (userA_gender, userA_school, userA_college,
                              userA_mbti, userB_gender, userB_school,
                              userB_college, userB_mbti)]

    grid = rows // rb
    blk = pl.BlockSpec((rb, 128), lambda i: (i, 0))
    out = pl.pallas_call(
        _lookup_body,
        out_shape=jax.ShapeDtypeStruct((rows, 128), jnp.float32),
        grid=(grid,),
        in_specs=[pl.BlockSpec((32, 128), lambda i: (0, 0))] + [blk] * 8,
        out_specs=blk,
        compiler_params=pltpu.CompilerParams(
            dimension_semantics=("parallel",),
            vmem_limit_bytes=32 << 20,
        ),
    )(tbl, *idxs)

    return out.reshape(-1)[:B].reshape(B, 1)


# speedup vs baseline: 8.6058x; 1.0408x over previous
import jax
import jax.numpy as jnp
from jax.experimental import pallas as pl
from jax.experimental.pallas import tpu as pltpu

_ROWS_PER_BLOCK = 512


def _pairwise_cos(emb, eps=1e-8):
    e = emb.astype(jnp.float32)
    dots = e @ e.T
    norms = jnp.sqrt(jnp.sum(e * e, axis=-1))
    return dots / jnp.maximum(norms[:, None] * norms[None, :], eps)


def _lookup_body(tbl_ref, idx_ref, out_ref):
    rb = out_ref.shape[0]
    rep = rb // 8
    small = pltpu.repeat(tbl_ref[0:8, :], rep, axis=0)
    m0 = pltpu.repeat(tbl_ref[8:16, :], rep, axis=0)
    m1 = pltpu.repeat(tbl_ref[16:24, :], rep, axis=0)
    m2 = pltpu.repeat(tbl_ref[24:32, :], rep, axis=0)

    p = idx_ref[...]
    kg = p & 3
    kc = (p >> 2) & 63
    ks = (p >> 8) & 127
    km = p >> 15
    kw = km & 127

    vg = jnp.take_along_axis(small, kg, axis=1)
    vc = jnp.take_along_axis(small, kc, axis=1)
    vs = jnp.take_along_axis(small, ks, axis=1)
    u0 = jnp.take_along_axis(m0, kw, axis=1)
    u1 = jnp.take_along_axis(m1, kw, axis=1)
    u2 = jnp.take_along_axis(m2, kw, axis=1)
    vm = jnp.where(km < 128, u0, jnp.where(km < 256, u1, u2))

    out_ref[...] = jax.nn.sigmoid(vg + vc + vs + vm)


def kernel(gender_emb, college_emb, school_emb, mbti_emb, final_fc_w,
           final_fc_b, weight,
           userA_gender, userA_school, userA_college, userA_mbti,
           userB_gender, userB_school, userB_college, userB_mbti):
    w_eff = (weight.astype(jnp.float32) *
             final_fc_w.reshape(4).astype(jnp.float32))
    bias = final_fc_b.astype(jnp.float32).reshape(())

    tg = (_pairwise_cos(gender_emb) * w_eff[0] + bias).reshape(-1)
    tc = (_pairwise_cos(college_emb) * w_eff[1]).reshape(-1)
    ts = (_pairwise_cos(school_emb) * w_eff[2]).reshape(-1)
    tm = (_pairwise_cos(mbti_emb) * w_eff[3]).reshape(-1)

    small = jnp.concatenate([tg, tc, ts, jnp.zeros((11,), jnp.float32)])
    mpad = jnp.concatenate([tm, jnp.zeros((384 - 289,), jnp.float32)])
    mchunks = mpad.reshape(3, 128)
    tbl = jnp.concatenate([
        jnp.tile(small[None, :], (8, 1)),
        jnp.tile(mchunks[0][None, :], (8, 1)),
        jnp.tile(mchunks[1][None, :], (8, 1)),
        jnp.tile(mchunks[2][None, :], (8, 1)),
    ], axis=0)

    B = userA_gender.shape[0]
    rb = _ROWS_PER_BLOCK
    rows = pl.cdiv(B, 128)
    rows = pl.cdiv(rows, rb) * rb
    bpad = rows * 128 - B

    ag, asc, aco, am, bg, bsc, bco, bm = (
        x.astype(jnp.int32) for x in (userA_gender, userA_school,
                                      userA_college, userA_mbti, userB_gender,
                                      userB_school, userB_college, userB_mbti))
    packed = ((ag * 2 + bg)
              | ((aco * 7 + bco + 4) << 2)
              | ((asc * 8 + bsc + 53) << 8)
              | ((am * 17 + bm) << 15))
    if bpad:
        packed = jnp.pad(packed, (0, bpad))
    packed = packed.reshape(rows, 128)

    grid = rows // rb
    blk = pl.BlockSpec((rb, 128), lambda i: (i, 0))
    out = pl.pallas_call(
        _lookup_body,
        out_shape=jax.ShapeDtypeStruct((rows, 128), jnp.float32),
        grid=(grid,),
        in_specs=[pl.BlockSpec((32, 128), lambda i: (0, 0)), blk],
        out_specs=blk,
        compiler_params=pltpu.CompilerParams(
            dimension_semantics=("parallel",),
            vmem_limit_bytes=32 << 20,
        ),
    )(tbl, packed)

    return out.reshape(-1)[:B].reshape(B, 1)
